# baseline (speedup 1.0000x reference)
import dataclasses
import functools
import math

import jax
import jax.numpy as jnp
from jax import lax
from jax.experimental import pallas as pl
from jax.experimental.pallas import tpu as pltpu
from jax.experimental.pallas import tpu_sc as plsc

B = 16
L = 576
SIZE = 24
D = 768
NUM_PARTS = 4
PARTS_DIM = D // NUM_PARTS
PART_CHANNELS = 32
GCN_DIM = 392
NUM_EDGES = NUM_PARTS * (NUM_PARTS - 1) * B
N_NODES = B * NUM_PARTS
BB = 2
PHA = B // BB
HID = PARTS_DIM // 16
CPS = 4
NKB = PART_CHANNELS // CPS
NTOT = PHA + NKB
PC4 = NUM_PARTS * PART_CHANNELS

_INV_SQRT2 = 1.0 / math.sqrt(2.0)


def _gelu(t):
    return 0.5 * t * (1.0 + jax.lax.erf(t * _INV_SQRT2))


def _ln(t, g, b, eps=1e-6):
    m = jnp.mean(t, axis=-1, keepdims=True)
    v = jnp.mean((t - m) ** 2, axis=-1, keepdims=True)
    return (t - m) / jnp.sqrt(v + eps) * g + b


def _dot(a, b):
    return jnp.dot(a, b, preferred_element_type=jnp.float32)


def _rsqrt16(x):
    seed = plsc.bitcast(
        jnp.int32(0x5F3759DF) - (plsc.bitcast(x, jnp.int32) >> 1), jnp.float32)
    r = seed
    for _ in range(3):
        r = r * (1.5 - 0.5 * x * r * r)
    return r


def _sc_adjacency(ei, ew0, ew1):
    mesh = plsc.VectorSubcoreMesh(core_axis_name="c", subcore_axis_name="s")
    NN = N_NODES * N_NODES
    cp = pltpu.CompilerParams()
    if "needs_layout_passes" in pltpu.CompilerParams.__dataclass_fields__:
        cp = dataclasses.replace(cp, needs_layout_passes=False)

    @functools.partial(
        pl.kernel, mesh=mesh, compiler_params=cp,
        out_type=jax.ShapeDtypeStruct((2, NN), jnp.float32),
        scratch_types=[
            pltpu.VMEM((NUM_EDGES,), jnp.int32),
            pltpu.VMEM((NUM_EDGES,), jnp.int32),
            pltpu.VMEM((NUM_EDGES,), jnp.int32),
            pltpu.VMEM((NUM_EDGES,), jnp.float32),
            pltpu.VMEM((NUM_EDGES,), jnp.float32),
            pltpu.VMEM((N_NODES,), jnp.float32),
            pltpu.VMEM((NN,), jnp.float32),
            pltpu.VMEM_SHARED((N_NODES,), jnp.float32),
            pltpu.VMEM_SHARED((NN,), jnp.float32),
            pltpu.SemaphoreType.DMA,
        ])
    def k(ei_hbm, ew0_hbm, ew1_hbm, out_hbm, src_v, dst_v, fi_v, sw_v, nv_v,
          deg_v, a_v, deg_s, a_s, sem):
        cid = lax.axis_index("c")
        sid = lax.axis_index("s")

        @pl.when(sid == 0)
        def _():
            pltpu.sync_copy(ei_hbm.at[0], src_v)
            pltpu.sync_copy(ei_hbm.at[1], dst_v)

            @pl.when(cid == 0)
            def _():
                pltpu.sync_copy(ew0_hbm, sw_v)

            @pl.when(cid == 1)
            def _():
                pltpu.sync_copy(ew1_hbm, sw_v)

            @pl.loop(0, N_NODES, step=16)
            def _(n0):
                deg_v[pl.ds(n0, 16)] = jnp.full((16,), 1.0, jnp.float32)

            @pl.loop(0, NN, step=16)
            def _(n0):
                a_v[pl.ds(n0, 16)] = jnp.zeros((16,), jnp.float32)

            pltpu.sync_copy(deg_v, deg_s)
            pltpu.sync_copy(a_v, a_s)

            @pl.loop(0, NUM_EDGES, step=16)
            def _(e0):
                w = sw_v[pl.ds(e0, 16)]
                sw_v[pl.ds(e0, 16)] = 1.0 / (1.0 + jnp.exp(-w))
                fi_v[pl.ds(e0, 16)] = (dst_v[pl.ds(e0, 16)] * N_NODES
                                       + src_v[pl.ds(e0, 16)])

            pltpu.sync_copy(sw_v, deg_s.at[dst_v], add=True)
            pltpu.sync_copy(deg_s, deg_v)

            @pl.loop(0, N_NODES, step=16)
            def _(n0):
                deg_v[pl.ds(n0, 16)] = _rsqrt16(deg_v[pl.ds(n0, 16)])

            @pl.loop(0, NUM_EDGES, step=16)
            def _(e0):
                ds = plsc.load_gather(deg_v, [src_v[pl.ds(e0, 16)]])
                dd = plsc.load_gather(deg_v, [dst_v[pl.ds(e0, 16)]])
                nv_v[pl.ds(e0, 16)] = ds * sw_v[pl.ds(e0, 16)] * dd

            pltpu.sync_copy(nv_v, a_s.at[fi_v], add=True)
            pltpu.sync_copy(a_s, a_v)

            @pl.loop(0, N_NODES, step=16)
            def _(n0):
                di = lax.iota(jnp.int32, 16) + n0
                fl = di * (N_NODES + 1)
                d = deg_v[pl.ds(n0, 16)]
                old = plsc.load_gather(a_v, [fl])
                plsc.store_scatter(a_v, [fl], old + d * d)

            pltpu.sync_copy(a_v, out_hbm.at[cid])

    return k(ei, ew0, ew1)


def _fused_body(x_ref, dm_ref, W1T_refs, W2T_refs, WpT_refs, bp_refs,
                ws_ref, bs_ref, fw_ref, fb_ref, A_ref,
                W1a_ref, b1a_ref, g1a_ref, be1a_ref,
                W2a_ref, b2a_ref, g2a_ref, be2a_ref,
                W1b_ref, b1b_ref, g1b_ref, be1b_ref,
                W2b_ref, b2b_ref, g2b_ref, be2b_ref,
                Wd_ref, gd_ref, bd_ref,
                m_ref, out_ref, x3_scr, sh_scr, acc_ref,
                W1bd_scr, W2bd_scr, Wpe_scr, bp_scr):
    i = pl.program_id(0)

    @pl.when(i == 0)
    def _():
        acc_ref[...] = jnp.zeros_like(acc_ref)
        W1bd_scr[...] = jnp.zeros_like(W1bd_scr)
        W2bd_scr[...] = jnp.zeros_like(W2bd_scr)
        Wpe_scr[...] = jnp.zeros_like(Wpe_scr)
        pd = jax.lax.broadcasted_iota(jnp.int32, (D, NUM_PARTS), 0)
        pc = jax.lax.broadcasted_iota(jnp.int32, (D, NUM_PARTS), 1)
        Wpe_scr[:, PC4:] = jnp.where(pd // PARTS_DIM == pc,
                                     1.0 / PARTS_DIM, 0.0)
        for p in range(NUM_PARTS):
            r0, r1 = p * PARTS_DIM, (p + 1) * PARTS_DIM
            W1bd_scr[r0:r1, p * HID:(p + 1) * HID] = W1T_refs[p][...].T
            W2bd_scr[p * HID:(p + 1) * HID, r0:r1] = W2T_refs[p][...].T
            Wpe_scr[r0:r1, p * PART_CHANNELS:(p + 1) * PART_CHANNELS] = (
                WpT_refs[p][...].T)
            bp_scr[0:1, p * PART_CHANNELS:(p + 1) * PART_CHANNELS] = (
                bp_refs[p][...][None, :])

    @pl.when(i < PHA)
    def _():
        for bb in range(BB):
            xf = x_ref[bb]
            dmv = dm_ref[bb].T
            avg_all = jnp.mean(xf, axis=0, keepdims=True)
            mx_all = jnp.max(xf, axis=0, keepdims=True)
            ha = jnp.maximum(_dot(avg_all, W1bd_scr[...]), 0.0)
            hm = jnp.maximum(_dot(mx_all, W1bd_scr[...]), 0.0)
            gate = jax.nn.sigmoid(_dot(ha, W2bd_scr[...])
                                  + _dot(hm, W2bd_scr[...]))
            xg = xf * gate
            z = _dot(xg, Wpe_scr[...])
            savg4 = z[:, PC4:]
            smax4 = jnp.concatenate(
                [jnp.max(xg[:, p * PARTS_DIM:(p + 1) * PARTS_DIM],
                         axis=1, keepdims=True) for p in range(NUM_PARTS)],
                axis=1)
            sm4 = jax.nn.sigmoid(savg4 * ws_ref[0:1] + smax4 * ws_ref[1:2]
                                 + dmv * ws_ref[2:3]
                                 + bs_ref[...][None, :])
            e32 = jnp.where(
                jax.lax.broadcasted_iota(jnp.int32, (NUM_PARTS, PC4), 0)
                == jax.lax.broadcasted_iota(jnp.int32, (NUM_PARTS, PC4), 1)
                // PART_CHANNELS, 1.0, 0.0)
            sm128 = _dot(sm4, e32)
            xo = z[:, :PC4] * sm128 + bp_scr[...]
            xoT = xo.T
            for p in range(NUM_PARTS):
                x3_scr[i, :, bb * NUM_PARTS + p, :] = (
                    xoT[p * PART_CHANNELS:(p + 1) * PART_CHANNELS])
            m_ref[bb] = sm4.T.reshape(NUM_PARTS, SIZE, SIZE)
            sh_scr[pl.ds(i * BB + bb, 1)] = avg_all

    @pl.when(i >= PHA)
    def _():
        c0 = (i - PHA) * CPS
        xgT = jnp.concatenate(
            [jnp.concatenate([x3_scr[s, c0 + j] for s in range(PHA)],
                             axis=0).T for j in range(CPS)], axis=0)
        acc_ref[...] += _dot(fw_ref[...], xgT)

    @pl.when(i == NTOT - 1)
    def _():
        y = acc_ref[...].T + fb_ref[...][None, :]
        A1 = A_ref[0]
        A2 = A_ref[1]

        h = _dot(A1, _dot(y, W1a_ref[...])) + b1a_ref[...]
        h = _gelu(_ln(h, g1a_ref[...], be1a_ref[...]))
        h = _dot(A1, _dot(h, W2a_ref[...])) + b2a_ref[...]
        y1 = _gelu(_ln(h, g2a_ref[...], be2a_ref[...]) + y)

        h = _dot(A2, _dot(y1, W1b_ref[...])) + b1b_ref[...]
        h = _gelu(_ln(h, g1b_ref[...], be1b_ref[...]))
        h = _dot(A2, _dot(h, W2b_ref[...])) + b2b_ref[...]
        h = _ln(h, g2b_ref[...], be2b_ref[...])
        sc = _ln(_dot(y1, Wd_ref[...]), gd_ref[...], bd_ref[...])
        y2 = _gelu(h + sc)

        pr = jax.lax.broadcasted_iota(jnp.int32, (B, N_NODES), 0)
        pc = jax.lax.broadcasted_iota(jnp.int32, (B, N_NODES), 1)
        pool = jnp.where(pc // NUM_PARTS == pr, 0.25, 0.0)
        out_ref[...] = _dot(pool, y2) + sh_scr[...]


@jax.jit
def kernel(decision_masks, x, params, edge_index):
    cb = params['cbam']
    ws_cols = jnp.stack([c['Ws'] for c in cb], axis=1)
    bs4 = jnp.stack([c['bs'] for c in cb])

    blocks = params['blocks']
    ei = edge_index.astype(jnp.int32)
    b0, b1 = blocks
    A2x = _sc_adjacency(ei, b0['edge_weight'], b1['edge_weight'])
    A2x = A2x.reshape(2, N_NODES, N_NODES)

    full = lambda s: pl.BlockSpec(s, lambda i: tuple(0 for _ in s))

    parts_masks, out = pl.pallas_call(
        _fused_body,
        grid=(NTOT,),
        in_specs=[
            pl.BlockSpec((BB, L, D), lambda i: (jnp.minimum(i, PHA - 1), 0, 0)),
            pl.BlockSpec((BB, 1, L), lambda i: (jnp.minimum(i, PHA - 1), 0, 0)),
            tuple(full((HID, PARTS_DIM)) for _ in range(NUM_PARTS)),
            tuple(full((PARTS_DIM, HID)) for _ in range(NUM_PARTS)),
            tuple(full((PART_CHANNELS, PARTS_DIM)) for _ in range(NUM_PARTS)),
            tuple(full((PART_CHANNELS,)) for _ in range(NUM_PARTS)),
            full((3, NUM_PARTS)),
            full((NUM_PARTS,)),
            pl.BlockSpec((GCN_DIM, CPS * L),
                         lambda i: (0, jnp.maximum(i - PHA, 0))),
            full((GCN_DIM,)),
            full((2, N_NODES, N_NODES)),
            full(b0['W1'].shape), full(b0['b1'].shape),
            full(b0['g1'].shape), full(b0['be1'].shape),
            full(b0['W2'].shape), full(b0['b2'].shape),
            full(b0['g2'].shape), full(b0['be2'].shape),
            full(b1['W1'].shape), full(b1['b1'].shape),
            full(b1['g1'].shape), full(b1['be1'].shape),
            full(b1['W2'].shape), full(b1['b2'].shape),
            full(b1['g2'].shape), full(b1['be2'].shape),
            full(b1['Wd'].shape), full(b1['gd'].shape),
            full(b1['bd'].shape),
        ],
        out_specs=[
            pl.BlockSpec((BB, NUM_PARTS, SIZE, SIZE),
                         lambda i: (jnp.minimum(i, PHA - 1), 0, 0, 0)),
            pl.BlockSpec((B, D), lambda i: (0, 0)),
        ],
        out_shape=[
            jax.ShapeDtypeStruct((B, NUM_PARTS, SIZE, SIZE), jnp.float32),
            jax.ShapeDtypeStruct((B, D), jnp.float32),
        ],
        scratch_shapes=[
            pltpu.VMEM((PHA, PART_CHANNELS, BB * NUM_PARTS, L), jnp.float32),
            pltpu.VMEM((B, D), jnp.float32),
            pltpu.VMEM((GCN_DIM, N_NODES), jnp.float32),
            pltpu.VMEM((D, NUM_PARTS * HID), jnp.float32),
            pltpu.VMEM((NUM_PARTS * HID, D), jnp.float32),
            pltpu.VMEM((D, PC4 + NUM_PARTS), jnp.float32),
            pltpu.VMEM((1, PC4), jnp.float32),
        ],
    )(x, decision_masks.transpose(0, 2, 1),
      tuple(c['W1'].T for c in cb),
      tuple(c['W2'].T for c in cb),
      tuple(c['Wp'].T for c in cb),
      tuple(c['bp'] for c in cb),
      ws_cols, bs4,
      params['fc_W'].T, params['fc_b'], A2x,
      b0['W1'], b0['b1'], b0['g1'], b0['be1'],
      b0['W2'], b0['b2'], b0['g2'], b0['be2'],
      b1['W1'], b1['b1'], b1['g1'], b1['be1'],
      b1['W2'], b1['b2'], b1['g2'], b1['be2'],
      b1['Wd'], b1['gd'], b1['bd'])

    return out, parts_masks
